# baseline (speedup 1.0000x reference)
import functools
import math

import jax
import jax.numpy as jnp
from jax import lax
from jax.experimental import pallas as pl
from jax.experimental.pallas import tpu as pltpu
from jax.experimental.pallas import tpu_sc as plsc

D_MODEL = 64
SCALE = math.sqrt(D_MODEL)
IDX_W = 128
K = 8
ROWS_PER_CHUNK = K * IDX_W
UNROLL = 8


def _build_sc_kernel(n_rows_idx):
    info = plsc.get_sparse_core_info()
    nc, ns = info.num_cores, info.num_subcores
    nw = nc * ns
    rows_per_worker = n_rows_idx // nw
    n_chunks = rows_per_worker // K
    b = n_rows_idx * IDX_W

    mesh = plsc.VectorSubcoreMesh(core_axis_name="c", subcore_axis_name="s")

    @functools.partial(
        pl.kernel,
        mesh=mesh,
        out_type=jax.ShapeDtypeStruct((b, D_MODEL), jnp.float32),
        scratch_types=[
            pltpu.VMEM((K, IDX_W), jnp.int32),
            pltpu.VMEM((ROWS_PER_CHUNK, D_MODEL), jnp.float32),
            pltpu.SemaphoreType.DMA,
        ],
        compiler_params=pltpu.CompilerParams(use_tc_tiling_on_sc=False),
    )
    def emb_kernel(table_hbm, idx_hbm, out_hbm, idx_v, rows_v, sem):
        wid = lax.axis_index("s") * nc + lax.axis_index("c")
        base_row = wid * rows_per_worker

        def chunk_body(g, carry):
            row0 = base_row + g * K
            pltpu.sync_copy(idx_hbm.at[pl.ds(row0, K)], idx_v)
            copies = []
            for j in range(K):
                copies.append(
                    pltpu.async_copy(
                        table_hbm.at[idx_v.at[j]],
                        rows_v.at[pl.ds(j * IDX_W, IDX_W)],
                        sem,
                    )
                )
            for c in copies:
                c.wait()

            def scale_body(i, carry2):
                r0 = i * UNROLL
                for rr in range(UNROLL):
                    for c4 in range(D_MODEL // 16):
                        sl = pl.ds(c4 * 16, 16)
                        rows_v[r0 + rr, sl] = rows_v[r0 + rr, sl] * SCALE
                return carry2

            lax.fori_loop(0, ROWS_PER_CHUNK // UNROLL, scale_body, 0,
                          unroll=False)

            pltpu.sync_copy(
                rows_v, out_hbm.at[pl.ds(row0 * IDX_W, ROWS_PER_CHUNK)]
            )
            return carry

        lax.fori_loop(0, n_chunks, chunk_body, 0, unroll=False)

    return emb_kernel


def kernel(x, emb_table):
    b = x.size
    xf = x.reshape(b // IDX_W, IDX_W).astype(jnp.int32)
    out = _build_sc_kernel(b // IDX_W)(emb_table, xf)
    return out.reshape(x.shape + (D_MODEL,))
